# baseline (speedup 1.0000x reference)
import functools

import jax
import jax.numpy as jnp
from jax import lax
from jax.experimental import pallas as pl
from jax.experimental.pallas import tpu as pltpu
from jax.experimental.pallas import tpu_sc as plsc

_N = 10000
_E = 320000
_D = 128
_DH = _D // 2
_NC = 2
_NS = 16
_B = 128
_NB = 160
_NP = 10240
_RPT = _NP // _NS
_DUMMY = _N
_EPAD = _NS * _NB * _B
_BLK = 1024
_G = _NP // _BLK

_mesh = plsc.VectorSubcoreMesh(core_axis_name="c", subcore_axis_name="s")
_SC_PARAMS = pltpu.CompilerParams(use_tc_tiling_on_sc=False)


def _zero_rows(ref, nrows, ncols16):
    z16 = jnp.zeros((16,), jnp.float32)

    def _row(i, carry):
        for j in range(ncols16):
            ref[i, pl.ds(j * 16, 16)] = z16
        return carry

    lax.fori_loop(0, nrows, _row, 0)


def _sc_common(c, s, hf_hbm, srci_hbm, dsti_hbm, acc_hbm,
               sidx, didx, bufs, acc_sh, gsems, ssems):
    pltpu.sync_copy(srci_hbm.at[c].at[s], sidx)
    pltpu.sync_copy(dsti_hbm.at[s], didx)

    _zero_rows(bufs[0], _B, _DH // 16)
    base = s * _RPT

    def _zacc(k, carry):
        pltpu.sync_copy(bufs[0], acc_sh.at[pl.ds(base + k * _B, _B)])
        return carry

    lax.fori_loop(0, _RPT // _B, _zacc, 0)
    plsc.subcore_barrier()

    nring = len(bufs)
    ng = _NB // nring
    for k in range(nring):
        pltpu.async_copy(hf_hbm.at[sidx.at[k]], bufs[k], gsems[k])

    def _main(g, carry):
        b0 = nring * g
        for k in range(nring):
            pltpu.make_async_copy(
                hf_hbm.at[sidx.at[b0 + k]], bufs[k], gsems[k]).wait()
            pltpu.async_copy(bufs[k], acc_sh.at[didx.at[b0 + k]], ssems[k],
                             add=True)

        @pl.when(g < ng - 1)
        def _():
            for k in range(nring):
                pltpu.make_async_copy(
                    bufs[k], acc_sh.at[didx.at[b0 + k]], ssems[k]).wait()
                pltpu.async_copy(
                    hf_hbm.at[sidx.at[b0 + nring + k]], bufs[k], gsems[k])

        return carry

    lax.fori_loop(0, ng, _main, 0)
    for k in range(nring):
        pltpu.make_async_copy(
            bufs[k], acc_sh.at[didx.at[_NB - nring + k]], ssems[k]).wait()
    plsc.subcore_barrier()

    def _wout(k, carry):
        r = base + k * _B
        pltpu.sync_copy(acc_sh.at[pl.ds(r, _B)], bufs[0])
        pltpu.sync_copy(bufs[0], acc_hbm.at[c].at[pl.ds(r, _B)])
        return carry

    lax.fori_loop(0, _RPT // _B, _wout, 0)


@functools.partial(
    pl.kernel,
    out_type=(
        jax.ShapeDtypeStruct((_NC, _NP, _DH), jnp.float32),
        jax.ShapeDtypeStruct((_NP, 16), jnp.float32),
    ),
    mesh=_mesh,
    scratch_types=[
        pltpu.VMEM((_NB, _B), jnp.int32),
        pltpu.VMEM((_NB, _B), jnp.int32),
        pltpu.VMEM((_B, _DH), jnp.float32),
        pltpu.VMEM((_B, _DH), jnp.float32),
        pltpu.VMEM((_B, 16), jnp.float32),
        pltpu.VMEM((_RPT, 16), jnp.float32),
        pltpu.VMEM_SHARED((_NP, _DH), jnp.float32),
        pltpu.VMEM_SHARED((_NP, 16), jnp.float32),
        pltpu.SemaphoreType.DMA,
        pltpu.SemaphoreType.DMA,
        pltpu.SemaphoreType.DMA,
        pltpu.SemaphoreType.DMA,
    ],
    compiler_params=_SC_PARAMS,
)
def _sc_agg_deg(hf_hbm, srci_hbm, dsti_hbm, acc_hbm, deg_hbm,
                sidx, didx, b0, b1, ones, zd, acc_sh, deg_sh,
                g0, g1, s0, s1):
    c = lax.axis_index("c")
    s = lax.axis_index("s")
    bufs = [b0, b1]

    @pl.when(c == 0)
    def _():
        _zero_rows(zd, _RPT, 1)
        o16 = jnp.ones((16,), jnp.float32)

        def _orow(i, carry):
            ones[i, :] = o16
            return carry

        lax.fori_loop(0, _B, _orow, 0)
        pltpu.sync_copy(zd, deg_sh.at[pl.ds(s * _RPT, _RPT)])

    _sc_common(c, s, hf_hbm, srci_hbm, dsti_hbm, acc_hbm,
               sidx, didx, bufs, acc_sh, (g0, g1), (s0, s1))

    @pl.when(c == 0)
    def _():
        def _degb(b, carry):
            pltpu.sync_copy(ones, deg_sh.at[didx.at[b]], add=True)
            return carry

        lax.fori_loop(0, _NB, _degb, 0)
        plsc.subcore_barrier()
        pltpu.sync_copy(deg_sh.at[pl.ds(s * _RPT, _RPT)], zd)
        pltpu.sync_copy(zd, deg_hbm.at[pl.ds(s * _RPT, _RPT)])


@functools.partial(
    pl.kernel,
    out_type=jax.ShapeDtypeStruct((_NC, _NP, _DH), jnp.float32),
    mesh=_mesh,
    scratch_types=[
        pltpu.VMEM((_NB, _B), jnp.int32),
        pltpu.VMEM((_NB, _B), jnp.int32),
        pltpu.VMEM((_B, _DH), jnp.float32),
        pltpu.VMEM((_B, _DH), jnp.float32),
        pltpu.VMEM((_B, _DH), jnp.float32),
        pltpu.VMEM((_B, _DH), jnp.float32),
        pltpu.VMEM_SHARED((_NP, _DH), jnp.float32),
        pltpu.SemaphoreType.DMA,
        pltpu.SemaphoreType.DMA,
        pltpu.SemaphoreType.DMA,
        pltpu.SemaphoreType.DMA,
        pltpu.SemaphoreType.DMA,
        pltpu.SemaphoreType.DMA,
        pltpu.SemaphoreType.DMA,
        pltpu.SemaphoreType.DMA,
    ],
    compiler_params=_SC_PARAMS,
)
def _sc_agg(hf_hbm, srci_hbm, dsti_hbm, acc_hbm,
            sidx, didx, b0, b1, b2, b3, acc_sh,
            g0, g1, g2, g3, s0, s1, s2, s3):
    c = lax.axis_index("c")
    s = lax.axis_index("s")
    bufs = [b0, b1, b2, b3]
    _sc_common(c, s, hf_hbm, srci_hbm, dsti_hbm, acc_hbm,
               sidx, didx, bufs, acc_sh, (g0, g1, g2, g3), (s0, s1, s2, s3))


def _tc1_body(h_ref, acc_ref, deg_ref, ws_ref, wn_ref, b_ref, out_ref, rd_ref):
    a = jnp.concatenate([acc_ref[0], acc_ref[1]], axis=-1)
    rd = 1.0 / jnp.maximum(deg_ref[:, 0:1], 1.0)
    mean = a * rd
    out = (jnp.dot(h_ref[...], ws_ref[...], preferred_element_type=jnp.float32)
           + jnp.dot(mean, wn_ref[...], preferred_element_type=jnp.float32)
           + b_ref[...])
    out_ref[...] = jnp.maximum(out, 0.0)
    rd_ref[...] = jnp.broadcast_to(rd, (_BLK, 16))


def _make_tc_body(relu):
    def _body(h_ref, acc_ref, rd_ref, ws_ref, wn_ref, b_ref, out_ref):
        a = jnp.concatenate([acc_ref[0], acc_ref[1]], axis=-1)
        mean = a * rd_ref[:, 0:1]
        out = (jnp.dot(h_ref[...], ws_ref[...], preferred_element_type=jnp.float32)
               + jnp.dot(mean, wn_ref[...], preferred_element_type=jnp.float32)
               + b_ref[...])
        out_ref[...] = jnp.maximum(out, 0.0) if relu else out
    return _body


_W_SPEC = pl.BlockSpec((_D, _D), lambda i: (0, 0))
_B_SPEC = pl.BlockSpec((1, _D), lambda i: (0, 0))
_H_SPEC = pl.BlockSpec((_BLK, _D), lambda i: (i, 0))
_ACC_SPEC = pl.BlockSpec((_NC, _BLK, _DH), lambda i: (0, i, 0))
_RD_SPEC = pl.BlockSpec((_BLK, 16), lambda i: (i, 0))

_tc_layer1 = pl.pallas_call(
    _tc1_body,
    grid=(_G,),
    in_specs=[_H_SPEC, _ACC_SPEC, _RD_SPEC, _W_SPEC, _W_SPEC, _B_SPEC],
    out_specs=[_H_SPEC, _RD_SPEC],
    out_shape=[jax.ShapeDtypeStruct((_NP, _D), jnp.float32),
               jax.ShapeDtypeStruct((_NP, 16), jnp.float32)],
)

_tc_layer_relu = pl.pallas_call(
    _make_tc_body(True),
    grid=(_G,),
    in_specs=[_H_SPEC, _ACC_SPEC, _RD_SPEC, _W_SPEC, _W_SPEC, _B_SPEC],
    out_specs=_H_SPEC,
    out_shape=jax.ShapeDtypeStruct((_NP, _D), jnp.float32),
)

_tc_layer_lin = pl.pallas_call(
    _make_tc_body(False),
    grid=(_G,),
    in_specs=[_H_SPEC, _ACC_SPEC, _RD_SPEC, _W_SPEC, _W_SPEC, _B_SPEC],
    out_specs=_H_SPEC,
    out_shape=jax.ShapeDtypeStruct((_NP, _D), jnp.float32),
)


def kernel(x, edge_index, edge_attr,
           Wself1, Wneigh1, b1, Wself2, Wneigh2, b2, Wself3, Wneigh3, b3,
           Wa, ba, W2a, b2a, Wb, bb, W2b, b2b):
    src = edge_index[0]
    dst = edge_index[1]
    pad = _EPAD - _E
    srcp = jnp.concatenate(
        [src, jnp.zeros((pad,), jnp.int32)]).reshape(_NS, _NB, _B)
    dstp = jnp.concatenate(
        [dst, jnp.full((pad,), _DUMMY, jnp.int32)]).reshape(_NS, _NB, _B)
    srcp2 = jnp.stack([2 * srcp, 2 * srcp + 1])
    xp = jnp.pad(x, ((0, _NP - _N), (0, 0)))

    acc1, deg = _sc_agg_deg(xp.reshape(2 * _NP, _DH), srcp2, dstp)
    h1, rdeg = _tc_layer1(xp, acc1, deg, Wself1, Wneigh1, b1.reshape(1, _D))
    acc2 = _sc_agg(h1.reshape(2 * _NP, _DH), srcp2, dstp)
    h2 = _tc_layer_relu(h1, acc2, rdeg, Wself2, Wneigh2, b2.reshape(1, _D))
    acc3 = _sc_agg(h2.reshape(2 * _NP, _DH), srcp2, dstp)
    h3 = _tc_layer_lin(h2, acc3, rdeg, Wself3, Wneigh3, b3.reshape(1, _D))
    return h3[:_N]


# speedup vs baseline: 1.6705x; 1.6705x over previous
import functools

import jax
import jax.numpy as jnp
from jax import lax
from jax.experimental import pallas as pl
from jax.experimental.pallas import tpu as pltpu
from jax.experimental.pallas import tpu_sc as plsc

_N = 10000
_E = 320000
_D = 128
_DQ = _D // 4
_NC = 2
_NS = 16
_B = 128
_NB = 160
_NP = 10240
_RPT = _NP // _NS
_DUMMY = _N
_EPAD = _NS * _NB * _B
_BLK = 1024
_G = _NP // _BLK
_NRING = 4

_mesh = plsc.VectorSubcoreMesh(core_axis_name="c", subcore_axis_name="s")
_SC_PARAMS = pltpu.CompilerParams(use_tc_tiling_on_sc=False)


def _zero_rows(ref, nrows, ncols16):
    z16 = jnp.zeros((16,), jnp.float32)

    def _row(i, carry):
        for j in range(ncols16):
            ref[i, pl.ds(j * 16, 16)] = z16
        return carry

    lax.fori_loop(0, nrows, _row, 0)


def _sc_phase(c, s, p, hq_hbm, acc_hbm, sidx, didx, bufs, tab_sh, acc_sh,
              gsems, ssems):
    base = s * _RPT
    q = c + 2 * p

    _zero_rows(bufs[0], _B, _DQ // 16)

    def _zacc(k, carry):
        pltpu.sync_copy(bufs[0], acc_sh.at[pl.ds(base + k * _B, _B)])
        return carry

    lax.fori_loop(0, _RPT // _B, _zacc, 0)

    def _stage(k, carry):
        r = base + k * _B
        pltpu.sync_copy(hq_hbm.at[q].at[pl.ds(r, _B)], bufs[1])
        pltpu.sync_copy(bufs[1], tab_sh.at[pl.ds(r, _B)])
        return carry

    lax.fori_loop(0, _RPT // _B, _stage, 0)
    plsc.subcore_barrier()

    for k in range(_NRING):
        pltpu.async_copy(tab_sh.at[sidx.at[k]], bufs[k], gsems[k])

    def _main(g, carry):
        b0 = _NRING * g
        for k in range(_NRING):
            pltpu.make_async_copy(
                tab_sh.at[sidx.at[b0 + k]], bufs[k], gsems[k]).wait()
            pltpu.async_copy(bufs[k], acc_sh.at[didx.at[b0 + k]], ssems[k],
                             add=True)

        @pl.when(g < _NB // _NRING - 1)
        def _():
            for k in range(_NRING):
                pltpu.make_async_copy(
                    bufs[k], acc_sh.at[didx.at[b0 + k]], ssems[k]).wait()
                pltpu.async_copy(
                    tab_sh.at[sidx.at[b0 + _NRING + k]], bufs[k], gsems[k])

        return carry

    lax.fori_loop(0, _NB // _NRING, _main, 0)
    for k in range(_NRING):
        pltpu.make_async_copy(
            bufs[k], acc_sh.at[didx.at[_NB - _NRING + k]], ssems[k]).wait()
    plsc.subcore_barrier()

    def _wout(k, carry):
        r = base + k * _B
        pltpu.sync_copy(acc_sh.at[pl.ds(r, _B)], bufs[0])
        pltpu.sync_copy(bufs[0], acc_hbm.at[q].at[pl.ds(r, _B)])
        return carry

    lax.fori_loop(0, _RPT // _B, _wout, 0)


def _sc_body(c, s, hq_hbm, srci_hbm, dsti_hbm, acc_hbm,
             sidx, didx, bufs, tab_sh, acc_sh, gsems, ssems):
    pltpu.sync_copy(srci_hbm.at[s], sidx)
    pltpu.sync_copy(dsti_hbm.at[s], didx)
    for p in range(2):
        _sc_phase(c, s, p, hq_hbm, acc_hbm, sidx, didx, bufs, tab_sh,
                  acc_sh, gsems, ssems)


_SC_SCRATCH = [
    pltpu.VMEM((_NB, _B), jnp.int32),
    pltpu.VMEM((_NB, _B), jnp.int32),
    pltpu.VMEM((_B, _DQ), jnp.float32),
    pltpu.VMEM((_B, _DQ), jnp.float32),
    pltpu.VMEM((_B, _DQ), jnp.float32),
    pltpu.VMEM((_B, _DQ), jnp.float32),
    pltpu.VMEM_SHARED((_NP, _DQ), jnp.float32),
    pltpu.VMEM_SHARED((_NP, _DQ), jnp.float32),
    pltpu.SemaphoreType.DMA,
    pltpu.SemaphoreType.DMA,
    pltpu.SemaphoreType.DMA,
    pltpu.SemaphoreType.DMA,
    pltpu.SemaphoreType.DMA,
    pltpu.SemaphoreType.DMA,
    pltpu.SemaphoreType.DMA,
    pltpu.SemaphoreType.DMA,
]


@functools.partial(
    pl.kernel,
    out_type=(
        jax.ShapeDtypeStruct((4, _NP, _DQ), jnp.float32),
        jax.ShapeDtypeStruct((_NP, 16), jnp.float32),
    ),
    mesh=_mesh,
    scratch_types=_SC_SCRATCH + [
        pltpu.VMEM((_B, 16), jnp.float32),
        pltpu.VMEM((_RPT, 16), jnp.float32),
        pltpu.VMEM_SHARED((_NP, 16), jnp.float32),
    ],
    compiler_params=_SC_PARAMS,
)
def _sc_agg_deg(hq_hbm, srci_hbm, dsti_hbm, acc_hbm, deg_hbm,
                sidx, didx, b0, b1, b2, b3, tab_sh, acc_sh,
                g0, g1, g2, g3, s0, s1, s2, s3, ones, zd, deg_sh):
    c = lax.axis_index("c")
    s = lax.axis_index("s")

    @pl.when(c == 0)
    def _():
        _zero_rows(zd, _RPT, 1)
        o16 = jnp.ones((16,), jnp.float32)

        def _orow(i, carry):
            ones[i, :] = o16
            return carry

        lax.fori_loop(0, _B, _orow, 0)
        pltpu.sync_copy(zd, deg_sh.at[pl.ds(s * _RPT, _RPT)])

    _sc_body(c, s, hq_hbm, srci_hbm, dsti_hbm, acc_hbm,
             sidx, didx, [b0, b1, b2, b3], tab_sh, acc_sh,
             (g0, g1, g2, g3), (s0, s1, s2, s3))

    @pl.when(c == 0)
    def _():
        def _degb(b, carry):
            pltpu.sync_copy(ones, deg_sh.at[didx.at[b]], add=True)
            return carry

        lax.fori_loop(0, _NB, _degb, 0)
        plsc.subcore_barrier()
        pltpu.sync_copy(deg_sh.at[pl.ds(s * _RPT, _RPT)], zd)
        pltpu.sync_copy(zd, deg_hbm.at[pl.ds(s * _RPT, _RPT)])


@functools.partial(
    pl.kernel,
    out_type=jax.ShapeDtypeStruct((4, _NP, _DQ), jnp.float32),
    mesh=_mesh,
    scratch_types=_SC_SCRATCH,
    compiler_params=_SC_PARAMS,
)
def _sc_agg(hq_hbm, srci_hbm, dsti_hbm, acc_hbm,
            sidx, didx, b0, b1, b2, b3, tab_sh, acc_sh,
            g0, g1, g2, g3, s0, s1, s2, s3):
    c = lax.axis_index("c")
    s = lax.axis_index("s")
    _sc_body(c, s, hq_hbm, srci_hbm, dsti_hbm, acc_hbm,
             sidx, didx, [b0, b1, b2, b3], tab_sh, acc_sh,
             (g0, g1, g2, g3), (s0, s1, s2, s3))


def _compute_out(h_ref, acc_ref, rd, ws_ref, wn_ref, b_ref):
    a = jnp.concatenate([acc_ref[0], acc_ref[1], acc_ref[2], acc_ref[3]],
                        axis=-1)
    mean = a * rd
    return (jnp.dot(h_ref[...], ws_ref[...], preferred_element_type=jnp.float32)
            + jnp.dot(mean, wn_ref[...], preferred_element_type=jnp.float32)
            + b_ref[...])


def _write_hq(hq_ref, out):
    for k in range(4):
        hq_ref[k] = out[:, k * _DQ:(k + 1) * _DQ]


def _tc1_body(h_ref, acc_ref, deg_ref, ws_ref, wn_ref, b_ref,
              out_ref, hq_ref, rd_ref):
    rd = 1.0 / jnp.maximum(deg_ref[:, 0:1], 1.0)
    out = jnp.maximum(_compute_out(h_ref, acc_ref, rd, ws_ref, wn_ref, b_ref),
                      0.0)
    out_ref[...] = out
    _write_hq(hq_ref, out)
    rd_ref[...] = jnp.broadcast_to(rd, (_BLK, 16))


def _tc2_body(h_ref, acc_ref, rd_ref, ws_ref, wn_ref, b_ref,
              out_ref, hq_ref):
    out = jnp.maximum(
        _compute_out(h_ref, acc_ref, rd_ref[:, 0:1], ws_ref, wn_ref, b_ref),
        0.0)
    out_ref[...] = out
    _write_hq(hq_ref, out)


def _tc3_body(h_ref, acc_ref, rd_ref, ws_ref, wn_ref, b_ref, out_ref):
    out_ref[...] = _compute_out(h_ref, acc_ref, rd_ref[:, 0:1],
                                ws_ref, wn_ref, b_ref)


_W_SPEC = pl.BlockSpec((_D, _D), lambda i: (0, 0))
_B_SPEC = pl.BlockSpec((1, _D), lambda i: (0, 0))
_H_SPEC = pl.BlockSpec((_BLK, _D), lambda i: (i, 0))
_Q_SPEC = pl.BlockSpec((4, _BLK, _DQ), lambda i: (0, i, 0))
_RD_SPEC = pl.BlockSpec((_BLK, 16), lambda i: (i, 0))

_tc_layer1 = pl.pallas_call(
    _tc1_body,
    grid=(_G,),
    in_specs=[_H_SPEC, _Q_SPEC, _RD_SPEC, _W_SPEC, _W_SPEC, _B_SPEC],
    out_specs=[_H_SPEC, _Q_SPEC, _RD_SPEC],
    out_shape=[jax.ShapeDtypeStruct((_NP, _D), jnp.float32),
               jax.ShapeDtypeStruct((4, _NP, _DQ), jnp.float32),
               jax.ShapeDtypeStruct((_NP, 16), jnp.float32)],
)

_tc_layer2 = pl.pallas_call(
    _tc2_body,
    grid=(_G,),
    in_specs=[_H_SPEC, _Q_SPEC, _RD_SPEC, _W_SPEC, _W_SPEC, _B_SPEC],
    out_specs=[_H_SPEC, _Q_SPEC],
    out_shape=[jax.ShapeDtypeStruct((_NP, _D), jnp.float32),
               jax.ShapeDtypeStruct((4, _NP, _DQ), jnp.float32)],
)

_tc_layer3 = pl.pallas_call(
    _tc3_body,
    grid=(_G,),
    in_specs=[_H_SPEC, _Q_SPEC, _RD_SPEC, _W_SPEC, _W_SPEC, _B_SPEC],
    out_specs=_H_SPEC,
    out_shape=jax.ShapeDtypeStruct((_NP, _D), jnp.float32),
)


def kernel(x, edge_index, edge_attr,
           Wself1, Wneigh1, b1, Wself2, Wneigh2, b2, Wself3, Wneigh3, b3,
           Wa, ba, W2a, b2a, Wb, bb, W2b, b2b):
    src = edge_index[0]
    dst = edge_index[1]
    pad = _EPAD - _E
    srcp = jnp.concatenate(
        [src, jnp.zeros((pad,), jnp.int32)]).reshape(_NS, _NB, _B)
    dstp = jnp.concatenate(
        [dst, jnp.full((pad,), _DUMMY, jnp.int32)]).reshape(_NS, _NB, _B)
    xp = jnp.pad(x, ((0, _NP - _N), (0, 0)))
    xq = xp.reshape(_NP, 4, _DQ).transpose(1, 0, 2)

    acc1, deg = _sc_agg_deg(xq, srcp, dstp)
    h1, hq1, rdeg = _tc_layer1(xp, acc1, deg, Wself1, Wneigh1,
                               b1.reshape(1, _D))
    acc2 = _sc_agg(hq1, srcp, dstp)
    h2, hq2 = _tc_layer2(h1, acc2, rdeg, Wself2, Wneigh2, b2.reshape(1, _D))
    acc3 = _sc_agg(hq2, srcp, dstp)
    h3 = _tc_layer3(h2, acc3, rdeg, Wself3, Wneigh3, b3.reshape(1, _D))
    return h3[:_N]


# speedup vs baseline: 1.7245x; 1.0323x over previous
import functools

import jax
import jax.numpy as jnp
from jax import lax
from jax.experimental import pallas as pl
from jax.experimental.pallas import tpu as pltpu
from jax.experimental.pallas import tpu_sc as plsc

_N = 10000
_E = 320000
_D = 128
_DQ = _D // 4
_NC = 2
_NS = 16
_B = 128
_NB = 160
_NP = 10240
_RPT = _NP // _NS
_DUMMY = _N
_EPAD = _NS * _NB * _B
_BLK = 1024
_G = _NP // _BLK
_NRING = 4

_mesh = plsc.VectorSubcoreMesh(core_axis_name="c", subcore_axis_name="s")
_SC_PARAMS = pltpu.CompilerParams(use_tc_tiling_on_sc=False)


def _zero_rows(ref, nrows, ncols16):
    z16 = jnp.zeros((16,), jnp.float32)

    def _row(i, carry):
        for j in range(ncols16):
            ref[i, pl.ds(j * 16, 16)] = z16
        return carry

    lax.fori_loop(0, nrows, _row, 0)


def _sc_phase(c, s, p, hq_hbm, acc_hbm, sidx, didx, bufs, tab_sh, acc_sh,
              gsems, ssems):
    base = s * _RPT
    q = c + 2 * p

    _zero_rows(bufs[0], _B, _DQ // 16)

    def _zacc(k, carry):
        pltpu.sync_copy(bufs[0], acc_sh.at[pl.ds(base + k * _B, _B)])
        return carry

    lax.fori_loop(0, _RPT // _B, _zacc, 0)

    pltpu.sync_copy(hq_hbm.at[q].at[pl.ds(base, _RPT)],
                    tab_sh.at[pl.ds(base, _RPT)])
    plsc.subcore_barrier()

    for k in range(_NRING):
        pltpu.async_copy(tab_sh.at[sidx.at[k]], bufs[k], gsems[k])

    def _main(g, carry):
        b0 = _NRING * g
        for k in range(_NRING):
            pltpu.make_async_copy(
                tab_sh.at[sidx.at[b0 + k]], bufs[k], gsems[k]).wait()
            pltpu.async_copy(bufs[k], acc_sh.at[didx.at[b0 + k]], ssems[k],
                             add=True)

        @pl.when(g < _NB // _NRING - 1)
        def _():
            for k in range(_NRING):
                pltpu.make_async_copy(
                    bufs[k], acc_sh.at[didx.at[b0 + k]], ssems[k]).wait()
                pltpu.async_copy(
                    tab_sh.at[sidx.at[b0 + _NRING + k]], bufs[k], gsems[k])

        return carry

    lax.fori_loop(0, _NB // _NRING, _main, 0)
    for k in range(_NRING):
        pltpu.make_async_copy(
            bufs[k], acc_sh.at[didx.at[_NB - _NRING + k]], ssems[k]).wait()
    plsc.subcore_barrier()

    pltpu.sync_copy(acc_sh.at[pl.ds(base, _RPT)],
                    acc_hbm.at[q].at[pl.ds(base, _RPT)])


def _sc_body(c, s, hq_hbm, srci_hbm, dsti_hbm, acc_hbm,
             sidx, didx, bufs, tab_sh, acc_sh, gsems, ssems):
    pltpu.sync_copy(srci_hbm.at[s], sidx)
    pltpu.sync_copy(dsti_hbm.at[s], didx)
    for p in range(2):
        _sc_phase(c, s, p, hq_hbm, acc_hbm, sidx, didx, bufs, tab_sh,
                  acc_sh, gsems, ssems)


_SC_SCRATCH = [
    pltpu.VMEM((_NB, _B), jnp.int32),
    pltpu.VMEM((_NB, _B), jnp.int32),
    pltpu.VMEM((_B, _DQ), jnp.float32),
    pltpu.VMEM((_B, _DQ), jnp.float32),
    pltpu.VMEM((_B, _DQ), jnp.float32),
    pltpu.VMEM((_B, _DQ), jnp.float32),
    pltpu.VMEM_SHARED((_NP, _DQ), jnp.float32),
    pltpu.VMEM_SHARED((_NP, _DQ), jnp.float32),
    pltpu.SemaphoreType.DMA,
    pltpu.SemaphoreType.DMA,
    pltpu.SemaphoreType.DMA,
    pltpu.SemaphoreType.DMA,
    pltpu.SemaphoreType.DMA,
    pltpu.SemaphoreType.DMA,
    pltpu.SemaphoreType.DMA,
    pltpu.SemaphoreType.DMA,
]


@functools.partial(
    pl.kernel,
    out_type=(
        jax.ShapeDtypeStruct((4, _NP, _DQ), jnp.float32),
        jax.ShapeDtypeStruct((_NP, 16), jnp.float32),
    ),
    mesh=_mesh,
    scratch_types=_SC_SCRATCH + [
        pltpu.VMEM((_B, 16), jnp.float32),
        pltpu.VMEM((_RPT, 16), jnp.float32),
        pltpu.VMEM_SHARED((_NP, 16), jnp.float32),
    ],
    compiler_params=_SC_PARAMS,
)
def _sc_agg_deg(hq_hbm, srci_hbm, dsti_hbm, acc_hbm, deg_hbm,
                sidx, didx, b0, b1, b2, b3, tab_sh, acc_sh,
                g0, g1, g2, g3, s0, s1, s2, s3, ones, zd, deg_sh):
    c = lax.axis_index("c")
    s = lax.axis_index("s")

    @pl.when(c == 0)
    def _():
        _zero_rows(zd, _RPT, 1)
        o16 = jnp.ones((16,), jnp.float32)

        def _orow(i, carry):
            ones[i, :] = o16
            return carry

        lax.fori_loop(0, _B, _orow, 0)
        pltpu.sync_copy(zd, deg_sh.at[pl.ds(s * _RPT, _RPT)])

    _sc_body(c, s, hq_hbm, srci_hbm, dsti_hbm, acc_hbm,
             sidx, didx, [b0, b1, b2, b3], tab_sh, acc_sh,
             (g0, g1, g2, g3), (s0, s1, s2, s3))

    @pl.when(c == 0)
    def _():
        def _degb(b, carry):
            pltpu.sync_copy(ones, deg_sh.at[didx.at[b]], add=True)
            return carry

        lax.fori_loop(0, _NB, _degb, 0)
        plsc.subcore_barrier()
        pltpu.sync_copy(deg_sh.at[pl.ds(s * _RPT, _RPT)], zd)
        pltpu.sync_copy(zd, deg_hbm.at[pl.ds(s * _RPT, _RPT)])


@functools.partial(
    pl.kernel,
    out_type=jax.ShapeDtypeStruct((4, _NP, _DQ), jnp.float32),
    mesh=_mesh,
    scratch_types=_SC_SCRATCH,
    compiler_params=_SC_PARAMS,
)
def _sc_agg(hq_hbm, srci_hbm, dsti_hbm, acc_hbm,
            sidx, didx, b0, b1, b2, b3, tab_sh, acc_sh,
            g0, g1, g2, g3, s0, s1, s2, s3):
    c = lax.axis_index("c")
    s = lax.axis_index("s")
    _sc_body(c, s, hq_hbm, srci_hbm, dsti_hbm, acc_hbm,
             sidx, didx, [b0, b1, b2, b3], tab_sh, acc_sh,
             (g0, g1, g2, g3), (s0, s1, s2, s3))


def _compute_out(h_ref, acc_ref, rd, ws_ref, wn_ref, b_ref):
    a = jnp.concatenate([acc_ref[0], acc_ref[1], acc_ref[2], acc_ref[3]],
                        axis=-1)
    mean = a * rd
    return (jnp.dot(h_ref[...], ws_ref[...], preferred_element_type=jnp.float32)
            + jnp.dot(mean, wn_ref[...], preferred_element_type=jnp.float32)
            + b_ref[...])


def _write_hq(hq_ref, out):
    for k in range(4):
        hq_ref[k] = out[:, k * _DQ:(k + 1) * _DQ]


def _tc1_body(h_ref, acc_ref, deg_ref, ws_ref, wn_ref, b_ref,
              out_ref, hq_ref, rd_ref):
    rd = 1.0 / jnp.maximum(deg_ref[:, 0:1], 1.0)
    out = jnp.maximum(_compute_out(h_ref, acc_ref, rd, ws_ref, wn_ref, b_ref),
                      0.0)
    out_ref[...] = out
    _write_hq(hq_ref, out)
    rd_ref[...] = jnp.broadcast_to(rd, (_BLK, 16))


def _tc2_body(h_ref, acc_ref, rd_ref, ws_ref, wn_ref, b_ref,
              out_ref, hq_ref):
    out = jnp.maximum(
        _compute_out(h_ref, acc_ref, rd_ref[:, 0:1], ws_ref, wn_ref, b_ref),
        0.0)
    out_ref[...] = out
    _write_hq(hq_ref, out)


def _tc3_body(h_ref, acc_ref, rd_ref, ws_ref, wn_ref, b_ref, out_ref):
    out_ref[...] = _compute_out(h_ref, acc_ref, rd_ref[:, 0:1],
                                ws_ref, wn_ref, b_ref)


_W_SPEC = pl.BlockSpec((_D, _D), lambda i: (0, 0))
_B_SPEC = pl.BlockSpec((1, _D), lambda i: (0, 0))
_H_SPEC = pl.BlockSpec((_BLK, _D), lambda i: (i, 0))
_Q_SPEC = pl.BlockSpec((4, _BLK, _DQ), lambda i: (0, i, 0))
_RD_SPEC = pl.BlockSpec((_BLK, 16), lambda i: (i, 0))

_tc_layer1 = pl.pallas_call(
    _tc1_body,
    grid=(_G,),
    in_specs=[_H_SPEC, _Q_SPEC, _RD_SPEC, _W_SPEC, _W_SPEC, _B_SPEC],
    out_specs=[_H_SPEC, _Q_SPEC, _RD_SPEC],
    out_shape=[jax.ShapeDtypeStruct((_NP, _D), jnp.float32),
               jax.ShapeDtypeStruct((4, _NP, _DQ), jnp.float32),
               jax.ShapeDtypeStruct((_NP, 16), jnp.float32)],
)

_tc_layer2 = pl.pallas_call(
    _tc2_body,
    grid=(_G,),
    in_specs=[_H_SPEC, _Q_SPEC, _RD_SPEC, _W_SPEC, _W_SPEC, _B_SPEC],
    out_specs=[_H_SPEC, _Q_SPEC],
    out_shape=[jax.ShapeDtypeStruct((_NP, _D), jnp.float32),
               jax.ShapeDtypeStruct((4, _NP, _DQ), jnp.float32)],
)

_tc_layer3 = pl.pallas_call(
    _tc3_body,
    grid=(_G,),
    in_specs=[_H_SPEC, _Q_SPEC, _RD_SPEC, _W_SPEC, _W_SPEC, _B_SPEC],
    out_specs=_H_SPEC,
    out_shape=jax.ShapeDtypeStruct((_NP, _D), jnp.float32),
)


def kernel(x, edge_index, edge_attr,
           Wself1, Wneigh1, b1, Wself2, Wneigh2, b2, Wself3, Wneigh3, b3,
           Wa, ba, W2a, b2a, Wb, bb, W2b, b2b):
    src = edge_index[0]
    dst = edge_index[1]
    pad = _EPAD - _E
    srcp = jnp.concatenate(
        [src, jnp.zeros((pad,), jnp.int32)]).reshape(_NS, _NB, _B)
    dstp = jnp.concatenate(
        [dst, jnp.full((pad,), _DUMMY, jnp.int32)]).reshape(_NS, _NB, _B)
    xp = jnp.pad(x, ((0, _NP - _N), (0, 0)))
    xq = xp.reshape(_NP, 4, _DQ).transpose(1, 0, 2)

    acc1, deg = _sc_agg_deg(xq, srcp, dstp)
    h1, hq1, rdeg = _tc_layer1(xp, acc1, deg, Wself1, Wneigh1,
                               b1.reshape(1, _D))
    acc2 = _sc_agg(hq1, srcp, dstp)
    h2, hq2 = _tc_layer2(h1, acc2, rdeg, Wself2, Wneigh2, b2.reshape(1, _D))
    acc3 = _sc_agg(hq2, srcp, dstp)
    h3 = _tc_layer3(h2, acc3, rdeg, Wself3, Wneigh3, b3.reshape(1, _D))
    return h3[:_N]


# speedup vs baseline: 1.7800x; 1.0322x over previous
import functools

import jax
import jax.numpy as jnp
from jax import lax
from jax.experimental import pallas as pl
from jax.experimental.pallas import tpu as pltpu
from jax.experimental.pallas import tpu_sc as plsc

_N = 10000
_E = 320000
_D = 128
_DQ = _D // 4
_NC = 2
_NS = 16
_B = 128
_NB = 160
_NP = 10240
_RPT = _NP // _NS
_DUMMY = _N
_EPAD = _NS * _NB * _B
_BLK = 1024
_G = _NP // _BLK
_NRING = 4

_mesh = plsc.VectorSubcoreMesh(core_axis_name="c", subcore_axis_name="s")
_SC_PARAMS = pltpu.CompilerParams(use_tc_tiling_on_sc=False)


def _zero_rows(ref, nrows, ncols16):
    z16 = jnp.zeros((16,), jnp.float32)

    def _row(i, carry):
        for j in range(ncols16):
            ref[i, pl.ds(j * 16, 16)] = z16
        return carry

    lax.fori_loop(0, nrows, _row, 0)


def _sc_phase(c, s, p, hq_hbm, acc_hbm, sidx, didx, bufs, tab_sh, acc_sh,
              gsems, ssems):
    base = s * _RPT
    q = c + 2 * p

    _zero_rows(bufs[0], _B, _DQ // 16)

    def _zacc(k, carry):
        pltpu.sync_copy(bufs[0], acc_sh.at[pl.ds(base + k * _B, _B)])
        return carry

    lax.fori_loop(0, _RPT // _B, _zacc, 0)

    pltpu.sync_copy(hq_hbm.at[q].at[pl.ds(base, _RPT)],
                    tab_sh.at[pl.ds(base, _RPT)])
    plsc.subcore_barrier()

    _NRING = len(bufs)
    for k in range(_NRING):
        pltpu.async_copy(tab_sh.at[sidx.at[k]], bufs[k], gsems[k])

    def _main(g, carry):
        b0 = _NRING * g
        for k in range(_NRING):
            pltpu.make_async_copy(
                tab_sh.at[sidx.at[b0 + k]], bufs[k], gsems[k]).wait()
            pltpu.async_copy(bufs[k], acc_sh.at[didx.at[b0 + k]], ssems[k],
                             add=True)

        @pl.when(g < _NB // _NRING - 1)
        def _():
            for k in range(_NRING):
                pltpu.make_async_copy(
                    bufs[k], acc_sh.at[didx.at[b0 + k]], ssems[k]).wait()
                pltpu.async_copy(
                    tab_sh.at[sidx.at[b0 + _NRING + k]], bufs[k], gsems[k])

        return carry

    lax.fori_loop(0, _NB // _NRING, _main, 0)
    for k in range(_NRING):
        pltpu.make_async_copy(
            bufs[k], acc_sh.at[didx.at[_NB - _NRING + k]], ssems[k]).wait()
    plsc.subcore_barrier()

    pltpu.sync_copy(acc_sh.at[pl.ds(base, _RPT)],
                    acc_hbm.at[q].at[pl.ds(base, _RPT)])


def _sc_body(c, s, hq_hbm, srci_hbm, dsti_hbm, acc_hbm,
             sidx, didx, bufs, tab_sh, acc_sh, gsems, ssems):
    pltpu.sync_copy(srci_hbm.at[s], sidx)
    pltpu.sync_copy(dsti_hbm.at[s], didx)
    for p in range(2):
        _sc_phase(c, s, p, hq_hbm, acc_hbm, sidx, didx, bufs, tab_sh,
                  acc_sh, gsems, ssems)


_SC_SCRATCH = [
    pltpu.VMEM((_NB, _B), jnp.int32),
    pltpu.VMEM((_NB, _B), jnp.int32),
    pltpu.VMEM((_B, _DQ), jnp.float32),
    pltpu.VMEM((_B, _DQ), jnp.float32),
    pltpu.VMEM((_B, _DQ), jnp.float32),
    pltpu.VMEM((_B, _DQ), jnp.float32),
    pltpu.VMEM_SHARED((_NP, _DQ), jnp.float32),
    pltpu.VMEM_SHARED((_NP, _DQ), jnp.float32),
    pltpu.SemaphoreType.DMA,
    pltpu.SemaphoreType.DMA,
    pltpu.SemaphoreType.DMA,
    pltpu.SemaphoreType.DMA,
    pltpu.SemaphoreType.DMA,
    pltpu.SemaphoreType.DMA,
    pltpu.SemaphoreType.DMA,
    pltpu.SemaphoreType.DMA,
]


@functools.partial(
    pl.kernel,
    out_type=(
        jax.ShapeDtypeStruct((4, _NP, _DQ), jnp.float32),
        jax.ShapeDtypeStruct((_NP, 16), jnp.float32),
    ),
    mesh=_mesh,
    scratch_types=_SC_SCRATCH + [
        pltpu.VMEM((_B, 16), jnp.float32),
        pltpu.VMEM((_RPT, 16), jnp.float32),
        pltpu.VMEM_SHARED((_NP, 16), jnp.float32),
    ],
    compiler_params=_SC_PARAMS,
)
def _sc_agg_deg(hq_hbm, srci_hbm, dsti_hbm, acc_hbm, deg_hbm,
                sidx, didx, b0, b1, b2, b3, tab_sh, acc_sh,
                g0, g1, g2, g3, s0, s1, s2, s3, ones, zd, deg_sh):
    c = lax.axis_index("c")
    s = lax.axis_index("s")

    @pl.when(c == 0)
    def _():
        _zero_rows(zd, _RPT, 1)
        o16 = jnp.ones((16,), jnp.float32)

        def _orow(i, carry):
            ones[i, :] = o16
            return carry

        lax.fori_loop(0, _B, _orow, 0)
        pltpu.sync_copy(zd, deg_sh.at[pl.ds(s * _RPT, _RPT)])

    _sc_body(c, s, hq_hbm, srci_hbm, dsti_hbm, acc_hbm,
             sidx, didx, [b0, b1, b2, b3], tab_sh, acc_sh,
             (g0, g1, g2, g3), (s0, s1, s2, s3))

    @pl.when(c == 0)
    def _():
        def _degb(b, carry):
            pltpu.sync_copy(ones, deg_sh.at[didx.at[b]], add=True)
            return carry

        lax.fori_loop(0, _NB, _degb, 0)
        plsc.subcore_barrier()
        pltpu.sync_copy(deg_sh.at[pl.ds(s * _RPT, _RPT)], zd)
        pltpu.sync_copy(zd, deg_hbm.at[pl.ds(s * _RPT, _RPT)])


@functools.partial(
    pl.kernel,
    out_type=jax.ShapeDtypeStruct((4, _NP, _DQ), jnp.float32),
    mesh=_mesh,
    scratch_types=_SC_SCRATCH + [pltpu.VMEM((_B, _DQ), jnp.float32)] * 4
    + [pltpu.SemaphoreType.DMA] * 8,
    compiler_params=_SC_PARAMS,
)
def _sc_agg(hq_hbm, srci_hbm, dsti_hbm, acc_hbm,
            sidx, didx, b0, b1, b2, b3, tab_sh, acc_sh,
            g0, g1, g2, g3, s0, s1, s2, s3,
            b4, b5, b6, b7, g4, g5, g6, g7, s4, s5, s6, s7):
    c = lax.axis_index("c")
    s = lax.axis_index("s")
    _sc_body(c, s, hq_hbm, srci_hbm, dsti_hbm, acc_hbm,
             sidx, didx, [b0, b1, b2, b3, b4, b5, b6, b7], tab_sh, acc_sh,
             (g0, g1, g2, g3, g4, g5, g6, g7),
             (s0, s1, s2, s3, s4, s5, s6, s7))


def _compute_out(h_ref, acc_ref, rd, ws_ref, wn_ref, b_ref):
    a = jnp.concatenate([acc_ref[0], acc_ref[1], acc_ref[2], acc_ref[3]],
                        axis=-1)
    mean = a * rd
    return (jnp.dot(h_ref[...], ws_ref[...], preferred_element_type=jnp.float32)
            + jnp.dot(mean, wn_ref[...], preferred_element_type=jnp.float32)
            + b_ref[...])


def _write_hq(hq_ref, out):
    for k in range(4):
        hq_ref[k] = out[:, k * _DQ:(k + 1) * _DQ]


def _tc1_body(h_ref, acc_ref, deg_ref, ws_ref, wn_ref, b_ref,
              out_ref, hq_ref, rd_ref):
    rd = 1.0 / jnp.maximum(deg_ref[:, 0:1], 1.0)
    out = jnp.maximum(_compute_out(h_ref, acc_ref, rd, ws_ref, wn_ref, b_ref),
                      0.0)
    out_ref[...] = out
    _write_hq(hq_ref, out)
    rd_ref[...] = jnp.broadcast_to(rd, (_BLK, 16))


def _tc2_body(h_ref, acc_ref, rd_ref, ws_ref, wn_ref, b_ref,
              out_ref, hq_ref):
    out = jnp.maximum(
        _compute_out(h_ref, acc_ref, rd_ref[:, 0:1], ws_ref, wn_ref, b_ref),
        0.0)
    out_ref[...] = out
    _write_hq(hq_ref, out)


def _tc3_body(h_ref, acc_ref, rd_ref, ws_ref, wn_ref, b_ref, out_ref):
    out_ref[...] = _compute_out(h_ref, acc_ref, rd_ref[:, 0:1],
                                ws_ref, wn_ref, b_ref)


_W_SPEC = pl.BlockSpec((_D, _D), lambda i: (0, 0))
_B_SPEC = pl.BlockSpec((1, _D), lambda i: (0, 0))
_H_SPEC = pl.BlockSpec((_BLK, _D), lambda i: (i, 0))
_Q_SPEC = pl.BlockSpec((4, _BLK, _DQ), lambda i: (0, i, 0))
_RD_SPEC = pl.BlockSpec((_BLK, 16), lambda i: (i, 0))

_tc_layer1 = pl.pallas_call(
    _tc1_body,
    grid=(_G,),
    in_specs=[_H_SPEC, _Q_SPEC, _RD_SPEC, _W_SPEC, _W_SPEC, _B_SPEC],
    out_specs=[_H_SPEC, _Q_SPEC, _RD_SPEC],
    out_shape=[jax.ShapeDtypeStruct((_NP, _D), jnp.float32),
               jax.ShapeDtypeStruct((4, _NP, _DQ), jnp.float32),
               jax.ShapeDtypeStruct((_NP, 16), jnp.float32)],
)

_tc_layer2 = pl.pallas_call(
    _tc2_body,
    grid=(_G,),
    in_specs=[_H_SPEC, _Q_SPEC, _RD_SPEC, _W_SPEC, _W_SPEC, _B_SPEC],
    out_specs=[_H_SPEC, _Q_SPEC],
    out_shape=[jax.ShapeDtypeStruct((_NP, _D), jnp.float32),
               jax.ShapeDtypeStruct((4, _NP, _DQ), jnp.float32)],
)

_tc_layer3 = pl.pallas_call(
    _tc3_body,
    grid=(_G,),
    in_specs=[_H_SPEC, _Q_SPEC, _RD_SPEC, _W_SPEC, _W_SPEC, _B_SPEC],
    out_specs=_H_SPEC,
    out_shape=jax.ShapeDtypeStruct((_NP, _D), jnp.float32),
)


def kernel(x, edge_index, edge_attr,
           Wself1, Wneigh1, b1, Wself2, Wneigh2, b2, Wself3, Wneigh3, b3,
           Wa, ba, W2a, b2a, Wb, bb, W2b, b2b):
    src = edge_index[0]
    dst = edge_index[1]
    pad = _EPAD - _E
    srcp = jnp.concatenate(
        [src, jnp.zeros((pad,), jnp.int32)]).reshape(_NS, _NB, _B)
    dstp = jnp.concatenate(
        [dst, jnp.full((pad,), _DUMMY, jnp.int32)]).reshape(_NS, _NB, _B)
    xp = jnp.pad(x, ((0, _NP - _N), (0, 0)))
    xq = xp.reshape(_NP, 4, _DQ).transpose(1, 0, 2)

    acc1, deg = _sc_agg_deg(xq, srcp, dstp)
    h1, hq1, rdeg = _tc_layer1(xp, acc1, deg, Wself1, Wneigh1,
                               b1.reshape(1, _D))
    acc2 = _sc_agg(hq1, srcp, dstp)
    h2, hq2 = _tc_layer2(h1, acc2, rdeg, Wself2, Wneigh2, b2.reshape(1, _D))
    acc3 = _sc_agg(hq2, srcp, dstp)
    h3 = _tc_layer3(h2, acc3, rdeg, Wself3, Wneigh3, b3.reshape(1, _D))
    return h3[:_N]


# speedup vs baseline: 1.8017x; 1.0122x over previous
import functools

import jax
import jax.numpy as jnp
from jax import lax
from jax.experimental import pallas as pl
from jax.experimental.pallas import tpu as pltpu
from jax.experimental.pallas import tpu_sc as plsc

_N = 10000
_E = 320000
_D = 128
_DQ = _D // 4
_NC = 2
_NS = 16
_B = 128
_NB = 160
_NP = 10240
_RPT = _NP // _NS
_DUMMY = _N
_EPAD = _NS * _NB * _B
_BLK = 1024
_G = _NP // _BLK
_NRING = 4

_mesh = plsc.VectorSubcoreMesh(core_axis_name="c", subcore_axis_name="s")
_SC_PARAMS = pltpu.CompilerParams(use_tc_tiling_on_sc=False)


def _zero_rows(ref, nrows, ncols16):
    z16 = jnp.zeros((16,), jnp.float32)

    def _row(i, carry):
        for j in range(ncols16):
            ref[i, pl.ds(j * 16, 16)] = z16
        return carry

    lax.fori_loop(0, nrows, _row, 0)


def _sc_phase(c, s, p, hq_hbm, acc_hbm, sidx, didx, bufs, tab_sh, acc_sh,
              gsems, ssems, degctx=None):
    base = s * _RPT
    q = c + 2 * p

    _zero_rows(bufs[0], _B, _DQ // 16)

    def _zacc(k, carry):
        pltpu.sync_copy(bufs[0], acc_sh.at[pl.ds(base + k * _B, _B)])
        return carry

    lax.fori_loop(0, _RPT // _B, _zacc, 0)

    pltpu.sync_copy(hq_hbm.at[q].at[pl.ds(base, _RPT)],
                    tab_sh.at[pl.ds(base, _RPT)])
    plsc.subcore_barrier()

    _NRING = len(bufs)
    for k in range(_NRING):
        pltpu.async_copy(tab_sh.at[sidx.at[k]], bufs[k], gsems[k])

    ndeg = _NB // _NC
    nper = ndeg // (_NB // _NRING)

    def _main(g, carry):
        b0 = _NRING * g
        if degctx is not None:
            ones, deg_sh, dsems = degctx
            d0 = c * ndeg + nper * g

            @pl.when(g > 0)
            def _():
                for j in range(nper):
                    pltpu.make_async_copy(
                        ones, deg_sh.at[didx.at[d0 - nper + j]],
                        dsems[j]).wait()

            for j in range(nper):
                pltpu.async_copy(ones, deg_sh.at[didx.at[d0 + j]], dsems[j],
                                 add=True)
        for k in range(_NRING):
            pltpu.make_async_copy(
                tab_sh.at[sidx.at[b0 + k]], bufs[k], gsems[k]).wait()
            pltpu.async_copy(bufs[k], acc_sh.at[didx.at[b0 + k]], ssems[k],
                             add=True)

        @pl.when(g < _NB // _NRING - 1)
        def _():
            for k in range(_NRING):
                pltpu.make_async_copy(
                    bufs[k], acc_sh.at[didx.at[b0 + k]], ssems[k]).wait()
                pltpu.async_copy(
                    tab_sh.at[sidx.at[b0 + _NRING + k]], bufs[k], gsems[k])

        return carry

    lax.fori_loop(0, _NB // _NRING, _main, 0)
    if degctx is not None:
        ones, deg_sh, dsems = degctx
        for j in range(nper):
            pltpu.make_async_copy(
                ones, deg_sh.at[didx.at[c * ndeg + ndeg - nper + j]],
                dsems[j]).wait()
    for k in range(_NRING):
        pltpu.make_async_copy(
            bufs[k], acc_sh.at[didx.at[_NB - _NRING + k]], ssems[k]).wait()
    plsc.subcore_barrier()

    pltpu.sync_copy(acc_sh.at[pl.ds(base, _RPT)],
                    acc_hbm.at[q].at[pl.ds(base, _RPT)])


def _sc_body(c, s, hq_hbm, srci_hbm, dsti_hbm, acc_hbm,
             sidx, didx, bufs, tab_sh, acc_sh, gsems, ssems, degctx=None):
    pltpu.sync_copy(srci_hbm.at[s], sidx)
    pltpu.sync_copy(dsti_hbm.at[s], didx)
    for p in range(2):
        _sc_phase(c, s, p, hq_hbm, acc_hbm, sidx, didx, bufs, tab_sh,
                  acc_sh, gsems, ssems, degctx=degctx if p == 0 else None)


_SC_SCRATCH = [
    pltpu.VMEM((_NB, _B), jnp.int32),
    pltpu.VMEM((_NB, _B), jnp.int32),
    pltpu.VMEM((_B, _DQ), jnp.float32),
    pltpu.VMEM((_B, _DQ), jnp.float32),
    pltpu.VMEM((_B, _DQ), jnp.float32),
    pltpu.VMEM((_B, _DQ), jnp.float32),
    pltpu.VMEM_SHARED((_NP, _DQ), jnp.float32),
    pltpu.VMEM_SHARED((_NP, _DQ), jnp.float32),
    pltpu.SemaphoreType.DMA,
    pltpu.SemaphoreType.DMA,
    pltpu.SemaphoreType.DMA,
    pltpu.SemaphoreType.DMA,
    pltpu.SemaphoreType.DMA,
    pltpu.SemaphoreType.DMA,
    pltpu.SemaphoreType.DMA,
    pltpu.SemaphoreType.DMA,
]


@functools.partial(
    pl.kernel,
    out_type=(
        jax.ShapeDtypeStruct((4, _NP, _DQ), jnp.float32),
        jax.ShapeDtypeStruct((_NC, _NP, 16), jnp.float32),
    ),
    mesh=_mesh,
    scratch_types=_SC_SCRATCH + [
        pltpu.VMEM((_B, 16), jnp.float32),
        pltpu.VMEM((_RPT, 16), jnp.float32),
        pltpu.VMEM_SHARED((_NP, 16), jnp.float32),
        pltpu.SemaphoreType.DMA,
        pltpu.SemaphoreType.DMA,
    ],
    compiler_params=_SC_PARAMS,
)
def _sc_agg_deg(hq_hbm, srci_hbm, dsti_hbm, acc_hbm, deg_hbm,
                sidx, didx, b0, b1, b2, b3, tab_sh, acc_sh,
                g0, g1, g2, g3, s0, s1, s2, s3, ones, zd, deg_sh, d0, d1):
    c = lax.axis_index("c")
    s = lax.axis_index("s")

    _zero_rows(zd, _RPT, 1)
    o16 = jnp.ones((16,), jnp.float32)

    def _orow(i, carry):
        ones[i, :] = o16
        return carry

    lax.fori_loop(0, _B, _orow, 0)
    pltpu.sync_copy(zd, deg_sh.at[pl.ds(s * _RPT, _RPT)])

    _sc_body(c, s, hq_hbm, srci_hbm, dsti_hbm, acc_hbm,
             sidx, didx, [b0, b1, b2, b3], tab_sh, acc_sh,
             (g0, g1, g2, g3), (s0, s1, s2, s3),
             degctx=(ones, deg_sh, (d0, d1)))

    pltpu.sync_copy(deg_sh.at[pl.ds(s * _RPT, _RPT)],
                    deg_hbm.at[c].at[pl.ds(s * _RPT, _RPT)])


@functools.partial(
    pl.kernel,
    out_type=jax.ShapeDtypeStruct((4, _NP, _DQ), jnp.float32),
    mesh=_mesh,
    scratch_types=_SC_SCRATCH + [pltpu.VMEM((_B, _DQ), jnp.float32)] * 4
    + [pltpu.SemaphoreType.DMA] * 8,
    compiler_params=_SC_PARAMS,
)
def _sc_agg(hq_hbm, srci_hbm, dsti_hbm, acc_hbm,
            sidx, didx, b0, b1, b2, b3, tab_sh, acc_sh,
            g0, g1, g2, g3, s0, s1, s2, s3,
            b4, b5, b6, b7, g4, g5, g6, g7, s4, s5, s6, s7):
    c = lax.axis_index("c")
    s = lax.axis_index("s")
    _sc_body(c, s, hq_hbm, srci_hbm, dsti_hbm, acc_hbm,
             sidx, didx, [b0, b1, b2, b3, b4, b5, b6, b7], tab_sh, acc_sh,
             (g0, g1, g2, g3, g4, g5, g6, g7),
             (s0, s1, s2, s3, s4, s5, s6, s7))


def _compute_out(h_ref, acc_ref, rd, ws_ref, wn_ref, b_ref):
    a = jnp.concatenate([acc_ref[0], acc_ref[1], acc_ref[2], acc_ref[3]],
                        axis=-1)
    mean = a * rd
    return (jnp.dot(h_ref[...], ws_ref[...], preferred_element_type=jnp.float32)
            + jnp.dot(mean, wn_ref[...], preferred_element_type=jnp.float32)
            + b_ref[...])


def _write_hq(hq_ref, out):
    for k in range(4):
        hq_ref[k] = out[:, k * _DQ:(k + 1) * _DQ]


def _tc1_body(h_ref, acc_ref, deg_ref, ws_ref, wn_ref, b_ref,
              out_ref, hq_ref, rd_ref):
    rd = 1.0 / jnp.maximum(deg_ref[0, :, 0:1] + deg_ref[1, :, 0:1], 1.0)
    out = jnp.maximum(_compute_out(h_ref, acc_ref, rd, ws_ref, wn_ref, b_ref),
                      0.0)
    out_ref[...] = out
    _write_hq(hq_ref, out)
    rd_ref[...] = jnp.broadcast_to(rd, (_BLK, 16))


def _tc2_body(h_ref, acc_ref, rd_ref, ws_ref, wn_ref, b_ref,
              out_ref, hq_ref):
    out = jnp.maximum(
        _compute_out(h_ref, acc_ref, rd_ref[:, 0:1], ws_ref, wn_ref, b_ref),
        0.0)
    out_ref[...] = out
    _write_hq(hq_ref, out)


def _tc3_body(h_ref, acc_ref, rd_ref, ws_ref, wn_ref, b_ref, out_ref):
    out_ref[...] = _compute_out(h_ref, acc_ref, rd_ref[:, 0:1],
                                ws_ref, wn_ref, b_ref)


_W_SPEC = pl.BlockSpec((_D, _D), lambda i: (0, 0))
_B_SPEC = pl.BlockSpec((1, _D), lambda i: (0, 0))
_H_SPEC = pl.BlockSpec((_BLK, _D), lambda i: (i, 0))
_Q_SPEC = pl.BlockSpec((4, _BLK, _DQ), lambda i: (0, i, 0))
_RD_SPEC = pl.BlockSpec((_BLK, 16), lambda i: (i, 0))

_DEG_SPEC = pl.BlockSpec((_NC, _BLK, 16), lambda i: (0, i, 0))

_tc_layer1 = pl.pallas_call(
    _tc1_body,
    grid=(_G,),
    in_specs=[_H_SPEC, _Q_SPEC, _DEG_SPEC, _W_SPEC, _W_SPEC, _B_SPEC],
    out_specs=[_H_SPEC, _Q_SPEC, _RD_SPEC],
    out_shape=[jax.ShapeDtypeStruct((_NP, _D), jnp.float32),
               jax.ShapeDtypeStruct((4, _NP, _DQ), jnp.float32),
               jax.ShapeDtypeStruct((_NP, 16), jnp.float32)],
)

_tc_layer2 = pl.pallas_call(
    _tc2_body,
    grid=(_G,),
    in_specs=[_H_SPEC, _Q_SPEC, _RD_SPEC, _W_SPEC, _W_SPEC, _B_SPEC],
    out_specs=[_H_SPEC, _Q_SPEC],
    out_shape=[jax.ShapeDtypeStruct((_NP, _D), jnp.float32),
               jax.ShapeDtypeStruct((4, _NP, _DQ), jnp.float32)],
)

_tc_layer3 = pl.pallas_call(
    _tc3_body,
    grid=(_G,),
    in_specs=[_H_SPEC, _Q_SPEC, _RD_SPEC, _W_SPEC, _W_SPEC, _B_SPEC],
    out_specs=_H_SPEC,
    out_shape=jax.ShapeDtypeStruct((_NP, _D), jnp.float32),
)


def kernel(x, edge_index, edge_attr,
           Wself1, Wneigh1, b1, Wself2, Wneigh2, b2, Wself3, Wneigh3, b3,
           Wa, ba, W2a, b2a, Wb, bb, W2b, b2b):
    src = edge_index[0]
    dst = edge_index[1]
    pad = _EPAD - _E
    srcp = jnp.concatenate(
        [src, jnp.zeros((pad,), jnp.int32)]).reshape(_NS, _NB, _B)
    dstp = jnp.concatenate(
        [dst, jnp.full((pad,), _DUMMY, jnp.int32)]).reshape(_NS, _NB, _B)
    xp = jnp.pad(x, ((0, _NP - _N), (0, 0)))
    xq = xp.reshape(_NP, 4, _DQ).transpose(1, 0, 2)

    acc1, deg = _sc_agg_deg(xq, srcp, dstp)
    h1, hq1, rdeg = _tc_layer1(xp, acc1, deg, Wself1, Wneigh1,
                               b1.reshape(1, _D))
    acc2 = _sc_agg(hq1, srcp, dstp)
    h2, hq2 = _tc_layer2(h1, acc2, rdeg, Wself2, Wneigh2, b2.reshape(1, _D))
    acc3 = _sc_agg(hq2, srcp, dstp)
    h3 = _tc_layer3(h2, acc3, rdeg, Wself3, Wneigh3, b3.reshape(1, _D))
    return h3[:_N]


# speedup vs baseline: 1.8049x; 1.0018x over previous
import functools

import jax
import jax.numpy as jnp
from jax import lax
from jax.experimental import pallas as pl
from jax.experimental.pallas import tpu as pltpu
from jax.experimental.pallas import tpu_sc as plsc

_N = 10000
_E = 320000
_D = 128
_DQ = _D // 4
_NC = 2
_NS = 16
_B = 128
_NB = 160
_NP = 10240
_RPT = _NP // _NS
_DUMMY = _N
_EPAD = _NS * _NB * _B
_BLK = 1024
_G = _NP // _BLK
_NRING = 4

_mesh = plsc.VectorSubcoreMesh(core_axis_name="c", subcore_axis_name="s")
_SC_PARAMS = pltpu.CompilerParams(use_tc_tiling_on_sc=False)


def _zero_rows(ref, nrows, ncols16):
    z16 = jnp.zeros((16,), jnp.float32)

    def _row(i, carry):
        for j in range(ncols16):
            ref[i, pl.ds(j * 16, 16)] = z16
        return carry

    lax.fori_loop(0, nrows, _row, 0)


def _sc_phase(c, s, p, hq_hbm, acc_hbm, sidx, didx, bufs, tab_sh, acc_sh,
              gsems, ssems, degctx=None):
    base = s * _RPT
    q = c + 2 * p

    _zero_rows(bufs[0], _B, _DQ // 16)

    def _zacc(k, carry):
        pltpu.sync_copy(bufs[0], acc_sh.at[pl.ds(base + k * _B, _B)])
        return carry

    lax.fori_loop(0, _RPT // _B, _zacc, 0)

    pltpu.sync_copy(hq_hbm.at[q].at[pl.ds(base, _RPT)],
                    tab_sh.at[pl.ds(base, _RPT)])
    plsc.subcore_barrier()

    _NRING = len(bufs)
    for k in range(_NRING):
        pltpu.async_copy(tab_sh.at[sidx.at[k]], bufs[k], gsems[k])

    ndeg = _NB // _NC
    nper = ndeg // (_NB // _NRING)

    def _main(g, carry):
        b0 = _NRING * g
        if degctx is not None:
            ones, deg_sh, dsems = degctx
            d0 = c * ndeg + nper * g

            @pl.when(g > 0)
            def _():
                for j in range(nper):
                    pltpu.make_async_copy(
                        ones, deg_sh.at[didx.at[d0 - nper + j]],
                        dsems[j]).wait()

            for j in range(nper):
                pltpu.async_copy(ones, deg_sh.at[didx.at[d0 + j]], dsems[j],
                                 add=True)
        for k in range(_NRING):
            pltpu.make_async_copy(
                tab_sh.at[sidx.at[b0 + k]], bufs[k], gsems[k]).wait()
            pltpu.async_copy(bufs[k], acc_sh.at[didx.at[b0 + k]], ssems[k],
                             add=True)

        @pl.when(g < _NB // _NRING - 1)
        def _():
            for k in range(_NRING):
                pltpu.make_async_copy(
                    bufs[k], acc_sh.at[didx.at[b0 + k]], ssems[k]).wait()
                pltpu.async_copy(
                    tab_sh.at[sidx.at[b0 + _NRING + k]], bufs[k], gsems[k])

        return carry

    lax.fori_loop(0, _NB // _NRING, _main, 0)
    if degctx is not None:
        ones, deg_sh, dsems = degctx
        for j in range(nper):
            pltpu.make_async_copy(
                ones, deg_sh.at[didx.at[c * ndeg + ndeg - nper + j]],
                dsems[j]).wait()
    for k in range(_NRING):
        pltpu.make_async_copy(
            bufs[k], acc_sh.at[didx.at[_NB - _NRING + k]], ssems[k]).wait()
    plsc.subcore_barrier()

    pltpu.sync_copy(acc_sh.at[pl.ds(base, _RPT)],
                    acc_hbm.at[q].at[pl.ds(base, _RPT)])


def _sc_body(c, s, hq_hbm, srci_hbm, dsti_hbm, acc_hbm,
             sidx, didx, bufs, tab_sh, acc_sh, gsems, ssems, degctx=None):
    pltpu.sync_copy(srci_hbm.at[s], sidx)
    pltpu.sync_copy(dsti_hbm.at[s], didx)
    for p in range(2):
        _sc_phase(c, s, p, hq_hbm, acc_hbm, sidx, didx, bufs, tab_sh,
                  acc_sh, gsems, ssems, degctx=degctx if p == 0 else None)


_SC_SCRATCH = [
    pltpu.VMEM((_NB, _B), jnp.int32),
    pltpu.VMEM((_NB, _B), jnp.int32),
    pltpu.VMEM((_B, _DQ), jnp.float32),
    pltpu.VMEM((_B, _DQ), jnp.float32),
    pltpu.VMEM((_B, _DQ), jnp.float32),
    pltpu.VMEM((_B, _DQ), jnp.float32),
    pltpu.VMEM_SHARED((_NP, _DQ), jnp.float32),
    pltpu.VMEM_SHARED((_NP, _DQ), jnp.float32),
    pltpu.SemaphoreType.DMA,
    pltpu.SemaphoreType.DMA,
    pltpu.SemaphoreType.DMA,
    pltpu.SemaphoreType.DMA,
    pltpu.SemaphoreType.DMA,
    pltpu.SemaphoreType.DMA,
    pltpu.SemaphoreType.DMA,
    pltpu.SemaphoreType.DMA,
]


@functools.partial(
    pl.kernel,
    out_type=(
        jax.ShapeDtypeStruct((4, _NP, _DQ), jnp.float32),
        jax.ShapeDtypeStruct((_NC, _NP, 16), jnp.float32),
    ),
    mesh=_mesh,
    scratch_types=_SC_SCRATCH + [
        pltpu.VMEM((_B, 16), jnp.float32),
        pltpu.VMEM((_RPT, 16), jnp.float32),
        pltpu.VMEM_SHARED((_NP, 16), jnp.float32),
        pltpu.SemaphoreType.DMA,
        pltpu.SemaphoreType.DMA,
    ],
    compiler_params=_SC_PARAMS,
)
def _sc_agg_deg(hq_hbm, srci_hbm, dsti_hbm, acc_hbm, deg_hbm,
                sidx, didx, b0, b1, b2, b3, tab_sh, acc_sh,
                g0, g1, g2, g3, s0, s1, s2, s3, ones, zd, deg_sh, d0, d1):
    c = lax.axis_index("c")
    s = lax.axis_index("s")

    _zero_rows(zd, _RPT, 1)
    o16 = jnp.ones((16,), jnp.float32)

    def _orow(i, carry):
        ones[i, :] = o16
        return carry

    lax.fori_loop(0, _B, _orow, 0)
    pltpu.sync_copy(zd, deg_sh.at[pl.ds(s * _RPT, _RPT)])

    _sc_body(c, s, hq_hbm, srci_hbm, dsti_hbm, acc_hbm,
             sidx, didx, [b0, b1, b2, b3], tab_sh, acc_sh,
             (g0, g1, g2, g3), (s0, s1, s2, s3),
             degctx=(ones, deg_sh, (d0, d1)))

    pltpu.sync_copy(deg_sh.at[pl.ds(s * _RPT, _RPT)],
                    deg_hbm.at[c].at[pl.ds(s * _RPT, _RPT)])


@functools.partial(
    pl.kernel,
    out_type=jax.ShapeDtypeStruct((4, _NP, _DQ), jnp.float32),
    mesh=_mesh,
    scratch_types=_SC_SCRATCH + [pltpu.VMEM((_B, _DQ), jnp.float32)] * 4
    + [pltpu.SemaphoreType.DMA] * 8,
    compiler_params=_SC_PARAMS,
)
def _sc_agg(hq_hbm, srci_hbm, dsti_hbm, acc_hbm,
            sidx, didx, b0, b1, b2, b3, tab_sh, acc_sh,
            g0, g1, g2, g3, s0, s1, s2, s3,
            b4, b5, b6, b7, g4, g5, g6, g7, s4, s5, s6, s7):
    c = lax.axis_index("c")
    s = lax.axis_index("s")
    _sc_body(c, s, hq_hbm, srci_hbm, dsti_hbm, acc_hbm,
             sidx, didx, [b0, b1, b2, b3, b4, b5, b6, b7], tab_sh, acc_sh,
             (g0, g1, g2, g3, g4, g5, g6, g7),
             (s0, s1, s2, s3, s4, s5, s6, s7))


def _compute_out(self_ref, acc_ref, rd, wn_ref):
    a = jnp.concatenate([acc_ref[0], acc_ref[1], acc_ref[2], acc_ref[3]],
                        axis=-1)
    mean = a * rd
    return (self_ref[...]
            + jnp.dot(mean, wn_ref[...], preferred_element_type=jnp.float32))


def _tc_self_body(h_ref, ws_ref, b_ref, out_ref):
    out_ref[...] = (
        jnp.dot(h_ref[...], ws_ref[...], preferred_element_type=jnp.float32)
        + b_ref[...])


def _write_hq(hq_ref, out):
    for k in range(4):
        hq_ref[k] = out[:, k * _DQ:(k + 1) * _DQ]


def _tc1_body(self_ref, acc_ref, deg_ref, wn_ref, out_ref, hq_ref, rd_ref):
    rd = 1.0 / jnp.maximum(deg_ref[0, :, 0:1] + deg_ref[1, :, 0:1], 1.0)
    out = jnp.maximum(_compute_out(self_ref, acc_ref, rd, wn_ref), 0.0)
    out_ref[...] = out
    _write_hq(hq_ref, out)
    rd_ref[...] = jnp.broadcast_to(rd, (_BLK, 16))


def _tc2_body(self_ref, acc_ref, rd_ref, wn_ref, out_ref, hq_ref):
    out = jnp.maximum(
        _compute_out(self_ref, acc_ref, rd_ref[:, 0:1], wn_ref), 0.0)
    out_ref[...] = out
    _write_hq(hq_ref, out)


def _tc3_body(self_ref, acc_ref, rd_ref, wn_ref, out_ref):
    out_ref[...] = _compute_out(self_ref, acc_ref, rd_ref[:, 0:1], wn_ref)


_W_SPEC = pl.BlockSpec((_D, _D), lambda i: (0, 0))
_B_SPEC = pl.BlockSpec((1, _D), lambda i: (0, 0))
_H_SPEC = pl.BlockSpec((_BLK, _D), lambda i: (i, 0))
_Q_SPEC = pl.BlockSpec((4, _BLK, _DQ), lambda i: (0, i, 0))
_RD_SPEC = pl.BlockSpec((_BLK, 16), lambda i: (i, 0))

_DEG_SPEC = pl.BlockSpec((_NC, _BLK, 16), lambda i: (0, i, 0))

_tc_self = pl.pallas_call(
    _tc_self_body,
    grid=(_G,),
    in_specs=[_H_SPEC, _W_SPEC, _B_SPEC],
    out_specs=_H_SPEC,
    out_shape=jax.ShapeDtypeStruct((_NP, _D), jnp.float32),
)

_tc_layer1 = pl.pallas_call(
    _tc1_body,
    grid=(_G,),
    in_specs=[_H_SPEC, _Q_SPEC, _DEG_SPEC, _W_SPEC],
    out_specs=[_H_SPEC, _Q_SPEC, _RD_SPEC],
    out_shape=[jax.ShapeDtypeStruct((_NP, _D), jnp.float32),
               jax.ShapeDtypeStruct((4, _NP, _DQ), jnp.float32),
               jax.ShapeDtypeStruct((_NP, 16), jnp.float32)],
)

_tc_layer2 = pl.pallas_call(
    _tc2_body,
    grid=(_G,),
    in_specs=[_H_SPEC, _Q_SPEC, _RD_SPEC, _W_SPEC],
    out_specs=[_H_SPEC, _Q_SPEC],
    out_shape=[jax.ShapeDtypeStruct((_NP, _D), jnp.float32),
               jax.ShapeDtypeStruct((4, _NP, _DQ), jnp.float32)],
)

_tc_layer3 = pl.pallas_call(
    _tc3_body,
    grid=(_G,),
    in_specs=[_H_SPEC, _Q_SPEC, _RD_SPEC, _W_SPEC],
    out_specs=_H_SPEC,
    out_shape=jax.ShapeDtypeStruct((_NP, _D), jnp.float32),
)


def kernel(x, edge_index, edge_attr,
           Wself1, Wneigh1, b1, Wself2, Wneigh2, b2, Wself3, Wneigh3, b3,
           Wa, ba, W2a, b2a, Wb, bb, W2b, b2b):
    src = edge_index[0]
    dst = edge_index[1]
    pad = _EPAD - _E
    srcp = jnp.concatenate(
        [src, jnp.zeros((pad,), jnp.int32)]).reshape(_NS, _NB, _B)
    dstp = jnp.concatenate(
        [dst, jnp.full((pad,), _DUMMY, jnp.int32)]).reshape(_NS, _NB, _B)
    xp = jnp.pad(x, ((0, _NP - _N), (0, 0)))
    xq = xp.reshape(_NP, 4, _DQ).transpose(1, 0, 2)

    acc1, deg = _sc_agg_deg(xq, srcp, dstp)
    self1 = _tc_self(xp, Wself1, b1.reshape(1, _D))
    h1, hq1, rdeg = _tc_layer1(self1, acc1, deg, Wneigh1)
    self2 = _tc_self(h1, Wself2, b2.reshape(1, _D))
    acc2 = _sc_agg(hq1, srcp, dstp)
    h2, hq2 = _tc_layer2(self2, acc2, rdeg, Wneigh2)
    self3 = _tc_self(h2, Wself3, b3.reshape(1, _D))
    acc3 = _sc_agg(hq2, srcp, dstp)
    h3 = _tc_layer3(self3, acc3, rdeg, Wneigh3)
    return h3[:_N]
